# Initial kernel scaffold; baseline (speedup 1.0000x reference)
import jax
import jax.numpy as jnp
from jax.experimental import pallas as pl


def kernel(x, nlDists, nlInds, pixels_h, pixels_w, both):
    raise NotImplementedError("write your pallas kernel here")


# speedup vs baseline: 134.3334x; 134.3334x over previous
import functools

import numpy as np

import jax
import jax.numpy as jnp
from jax import lax
from jax.experimental import pallas as pl
from jax.experimental.pallas import tpu as pltpu
from jax.experimental.pallas import tpu_sc as plsc

_T, _P, _C, _PS = 2, 65536, 3, 7
_HP, _WP = 256, 256
_N = _T * _P
_NPIX = _HP * _WP
_VIDN = _T * _C * _NPIX
_WN = _T * _NPIX
_PP = _PS * _PS
_PATCH = _C * _PP

_NC, _NS = 2, 16
_NWORK = _NC * _NS
_PPW = _N // _NWORK
_CHUNK = 128
_NCHUNK = _PPW // _CHUNK
_CV = _CHUNK * _PATCH
_CW = _CHUNK * _PP

_VSTRIPE = _VIDN // _NS
_WSTRIPE = _WN // _NS

_mesh = plsc.VectorSubcoreMesh(core_axis_name="c", subcore_axis_name="s")


_VSTARTS = (0, 16, 32, 48, 64, 80, 96, 112, 128, 131)
_WSTARTS = (0, 16, 32, 33)


def _host_offsets():
    offj = np.array([c * _NPIX + dy * _WP + dx
                     for c in range(_C)
                     for dy in range(_PS) for dx in range(_PS)], np.int32)
    offv = np.concatenate([offj[s:s + 16] for s in _VSTARTS])
    offjw = np.array([dy * _WP + dx
                      for dy in range(_PS) for dx in range(_PS)], np.int32)
    offw = np.concatenate([offjw[s:s + 16] for s in _WSTARTS])
    return offv, offw


_OFFV_TAB, _OFFW_TAB = _host_offsets()


def _gen_indices(idxv, basesv, offv, tbuf, hbuf, wbuf,
                 idxw=None, basesw=None, offw=None):
    def pb(e, carry):
        sl = pl.ds(e * 16, 16)
        t_v = tbuf[sl]
        pix = hbuf[sl] * _WP + wbuf[sl]
        basesv[sl] = t_v * (_C * _NPIX) + pix
        if basesw is not None:
            basesw[sl] = t_v * _NPIX + pix
        return carry
    lax.fori_loop(0, _CHUNK // 16, pb, 0)

    def pp(e, carry):
        bvv = basesv[pl.ds(e * 16, 16)]
        bwv = basesw[pl.ds(e * 16, 16)] if idxw is not None else None
        for lane in range(16):
            bv = bvv[lane]
            s0 = (e * 16 + lane) * _PATCH
            for si, s in enumerate(_VSTARTS):
                idxv[pl.ds(s0 + s, 16)] = offv[pl.ds(si * 16, 16)] + bv
            if idxw is not None:
                bw = bwv[lane]
                sw = (e * 16 + lane) * _PP
                for si, s in enumerate(_WSTARTS):
                    idxw[pl.ds(sw + s, 16)] = offw[pl.ds(si * 16, 16)] + bw
        return carry
    lax.fori_loop(0, _CHUNK // 16, pp, 0)


@functools.partial(
    pl.kernel,
    out_type=(jax.ShapeDtypeStruct((_NC, _VIDN), jnp.float32),
              jax.ShapeDtypeStruct((_NC, _WN), jnp.float32)),
    mesh=_mesh,
    scratch_types=[
        pltpu.VMEM_SHARED((_VIDN,), jnp.float32),
        pltpu.VMEM_SHARED((_WN,), jnp.float32),
        pltpu.VMEM((_CV,), jnp.float32),
        pltpu.VMEM((_CW,), jnp.float32),
        pltpu.VMEM((_CV,), jnp.int32),
        pltpu.VMEM((_CW,), jnp.int32),
        pltpu.VMEM((_CHUNK,), jnp.int32),
        pltpu.VMEM((_CHUNK,), jnp.int32),
        pltpu.VMEM((_CHUNK,), jnp.int32),
        pltpu.VMEM((_CHUNK,), jnp.int32),
        pltpu.VMEM((_CHUNK,), jnp.int32),
        pltpu.VMEM((len(_VSTARTS) * 16,), jnp.int32),
        pltpu.VMEM((len(_WSTARTS) * 16,), jnp.int32),
        pltpu.VMEM((2048,), jnp.float32),
    ],
)
def _fold(xf, tiv, hiv, wiv, offv_h, offw_h, vid_out, w_out, vid_sh, w_sh,
          xbuf, ones, idxv, idxw, tbuf, hbuf, wbuf, basesv, basesw,
          offv, offw, zbuf):
    core = lax.axis_index("c")
    sub = lax.axis_index("s")
    wid = core * _NS + sub
    zero16 = jnp.zeros((16,), jnp.float32)
    one16 = jnp.ones((16,), jnp.float32)

    def zb(i, carry):
        zbuf[pl.ds(i * 16, 16)] = zero16
        return carry
    lax.fori_loop(0, 2048 // 16, zb, 0)

    def ob(i, carry):
        ones[pl.ds(i * 16, 16)] = one16
        return carry
    lax.fori_loop(0, _CW // 16, ob, 0)

    voff = sub * _VSTRIPE
    woff = sub * _WSTRIPE

    def zv(i, carry):
        pltpu.sync_copy(zbuf, vid_sh.at[pl.ds(voff + i * 2048, 2048)])
        return carry
    lax.fori_loop(0, _VSTRIPE // 2048, zv, 0)

    def zw(i, carry):
        pltpu.sync_copy(zbuf, w_sh.at[pl.ds(woff + i * 2048, 2048)])
        return carry
    lax.fori_loop(0, _WSTRIPE // 2048, zw, 0)

    pltpu.sync_copy(offv_h, offv)
    pltpu.sync_copy(offw_h, offw)
    plsc.subcore_barrier()

    def chunk_body(i, carry):
        pbase = wid * _PPW + i * _CHUNK
        pltpu.sync_copy(tiv.at[pl.ds(pbase, _CHUNK)], tbuf)
        pltpu.sync_copy(hiv.at[pl.ds(pbase, _CHUNK)], hbuf)
        pltpu.sync_copy(wiv.at[pl.ds(pbase, _CHUNK)], wbuf)
        pltpu.sync_copy(xf.at[pl.ds(pbase * _PATCH, _CV)], xbuf)
        _gen_indices(idxv, basesv, offv, tbuf, hbuf, wbuf,
                     idxw=idxw, basesw=basesw, offw=offw)
        pltpu.sync_copy(xbuf, vid_sh.at[idxv], add=True)
        pltpu.sync_copy(ones, w_sh.at[idxw], add=True)
        return carry
    lax.fori_loop(0, _NCHUNK, chunk_body, 0)

    plsc.subcore_barrier()
    pltpu.sync_copy(vid_sh.at[pl.ds(voff, _VSTRIPE)],
                    vid_out.at[core, pl.ds(voff, _VSTRIPE)])
    pltpu.sync_copy(w_sh.at[pl.ds(woff, _WSTRIPE)],
                    w_out.at[core, pl.ds(woff, _WSTRIPE)])


@functools.partial(
    pl.kernel,
    out_type=jax.ShapeDtypeStruct((_N * _PATCH,), jnp.float32),
    mesh=_mesh,
    scratch_types=[
        pltpu.VMEM_SHARED((_VIDN,), jnp.float32),
        pltpu.VMEM((2048,), jnp.float32),
        pltpu.VMEM((2048,), jnp.float32),
        pltpu.VMEM((2048,), jnp.float32),
        pltpu.VMEM((2048,), jnp.float32),
        pltpu.VMEM((_CV,), jnp.float32),
        pltpu.VMEM((_CV,), jnp.int32),
        pltpu.VMEM((_CHUNK,), jnp.int32),
        pltpu.VMEM((_CHUNK,), jnp.int32),
        pltpu.VMEM((_CHUNK,), jnp.int32),
        pltpu.VMEM((_CHUNK,), jnp.int32),
        pltpu.VMEM((len(_VSTARTS) * 16,), jnp.int32),
    ],
)
def _unfold(vidp, wvp, tiv, hiv, wiv, offv_h, out, img_sh,
            v0, v1, w0, w1, obuf, idxv, tbuf, hbuf, wbuf, basesv, offv):
    core = lax.axis_index("c")
    sub = lax.axis_index("s")
    wid = core * _NS + sub

    pltpu.sync_copy(offv_h, offv)

    def blk(b, carry):
        g = sub * _VSTRIPE + b * 2048
        plane = lax.shift_right_logical(g, 16)
        t = jnp.where(plane >= _C, 1, 0)
        wg = t * _NPIX + (g - plane * _NPIX)
        pltpu.sync_copy(vidp.at[0, pl.ds(g, 2048)], v0)
        pltpu.sync_copy(vidp.at[1, pl.ds(g, 2048)], v1)
        pltpu.sync_copy(wvp.at[0, pl.ds(wg, 2048)], w0)
        pltpu.sync_copy(wvp.at[1, pl.ds(wg, 2048)], w1)

        def cb(i, c2):
            sl = pl.ds(i * 16, 16)
            num = v0[sl] + v1[sl]
            den = jnp.maximum(w0[sl] + w1[sl], 1e-10)
            v0[sl] = num / den
            return c2
        lax.fori_loop(0, 2048 // 16, cb, 0)
        pltpu.sync_copy(v0, img_sh.at[pl.ds(g, 2048)])
        return carry
    lax.fori_loop(0, _VSTRIPE // 2048, blk, 0)

    plsc.subcore_barrier()

    def chunk_body(i, carry):
        pbase = wid * _PPW + i * _CHUNK
        pltpu.sync_copy(tiv.at[pl.ds(pbase, _CHUNK)], tbuf)
        pltpu.sync_copy(hiv.at[pl.ds(pbase, _CHUNK)], hbuf)
        pltpu.sync_copy(wiv.at[pl.ds(pbase, _CHUNK)], wbuf)
        _gen_indices(idxv, basesv, offv, tbuf, hbuf, wbuf)
        pltpu.sync_copy(img_sh.at[idxv], obuf)
        pltpu.sync_copy(obuf, out.at[pl.ds(pbase * _PATCH, _CV)])
        return carry
    lax.fori_loop(0, _NCHUNK, chunk_body, 0)


def kernel(x, nlDists, nlInds, pixels_h, pixels_w, both):
    xf = x.reshape(_N * _PATCH)
    inds = nlInds[:, :, 0, :].reshape(_N, 3).astype(jnp.int32)
    tiv = inds[:, 0]
    hiv = inds[:, 1]
    wiv = inds[:, 2]
    offv_h = jnp.asarray(_OFFV_TAB)
    offw_h = jnp.asarray(_OFFW_TAB)
    vidp, wvp = _fold(xf, tiv, hiv, wiv, offv_h, offw_h)
    outf = _unfold(vidp, wvp, tiv, hiv, wiv, offv_h)
    return outf.reshape(_T, _P, 1, _PATCH)
